# baseline (speedup 1.0000x reference)
import functools

import jax
import jax.numpy as jnp
from jax.experimental import pallas as pl

_REG = 17
_C = 80
_K = 300
_IOU_THR = 0.7
_NEG = float("-inf")


def _reduce_kernel(cls_ref, out_ref):
    out_ref[0] = jnp.max(cls_ref[...], axis=-1)


def _sortable(x):
    xi = jax.lax.bitcast_convert_type(x, jnp.int32)
    return jnp.where(xi >= 0, xi, xi ^ jnp.int32(0x7FFFFFFF))


def _cumsum_lanes(x):
    b, l = x.shape
    d = 1
    while d < l:
        x = x + jnp.concatenate(
            [jnp.zeros((b, d), x.dtype), x[:, :l - d]], axis=1)
        d *= 2
    return x


def _select_kernel(m_ref, sel_ref, slot_ref):
    b, l = m_ref.shape
    x = m_ref[...]
    s = _sortable(x)
    key = s ^ jnp.int32(-2147483648)

    p = jnp.zeros((b, 1), jnp.int32)
    r = jnp.full((b, 1), _K, jnp.int32)
    for i in range(31, -1, -1):
        hi = jax.lax.shift_right_logical(key, jnp.int32(i))
        want = jax.lax.shift_right_logical(p, jnp.int32(i)) | jnp.int32(1)
        cnt = jnp.sum(jnp.where(hi == want, 1.0, 0.0), axis=1,
                      keepdims=True).astype(jnp.int32)
        take_hi = cnt >= r
        bit = jnp.int32(-2147483648) if i == 31 else jnp.int32(1 << i)
        p = jnp.where(take_hi, p | bit, p)
        r = jnp.where(take_hi, r, r - cnt)

    t = p ^ jnp.int32(-2147483648)
    gt = s > t
    n_gt = jnp.sum(jnp.where(gt, 1.0, 0.0), axis=1, keepdims=True)
    tie = s == t
    tiepos = _cumsum_lanes(jnp.where(tie, 1.0, 0.0))
    sel = gt | (tie & (tiepos <= (_K - n_gt)))
    self32 = jnp.where(sel, 1.0, 0.0)
    sel_ref[...] = self32
    slot_ref[...] = _cumsum_lanes(self32) - 1.0


def _decode_nms_kernel(pred_ref, m_ref, sel_ref, slot_ref, anch_ref,
                       str_ref, proj_ref, out_ref):
    l = pred_ref.shape[1]
    sel_row = sel_ref[0]
    slot_row = slot_ref[0]
    m_row = m_ref[0]
    str_row = str_ref[...]

    nchunk = 4
    ch = l // nchunk
    acc = jnp.zeros((_K, 4 * _REG), jnp.float32)
    acc_v = jnp.zeros((_K, 1), jnp.float32)
    acc_x = jnp.zeros((_K, 1), jnp.float32)
    acc_y = jnp.zeros((_K, 1), jnp.float32)
    acc_s = jnp.zeros((_K, 1), jnp.float32)
    kcol = jax.lax.broadcasted_iota(jnp.int32, (_K, ch), 0).astype(jnp.float32)
    for c in range(nchunk):
        cs = slice(c * ch, (c + 1) * ch)
        oh = jnp.where((slot_row[:, cs] == kcol)
                       & (sel_row[:, cs] > 0.5), 1.0, 0.0)
        acc = acc + jnp.dot(oh, pred_ref[0, cs, :],
                            preferred_element_type=jnp.float32)
        acc_v = acc_v + jnp.sum(oh * m_row[:, cs], axis=1, keepdims=True)
        acc_x = acc_x + jnp.sum(oh * anch_ref[0:1, cs], axis=1, keepdims=True)
        acc_y = acc_y + jnp.sum(oh * anch_ref[1:2, cs], axis=1, keepdims=True)
        acc_s = acc_s + jnp.sum(oh * str_row[:, cs], axis=1, keepdims=True)

    v = acc_v
    vr = jnp.reshape(v, (1, _K))
    ri = jax.lax.broadcasted_iota(jnp.int32, (_K, _K), 0)
    ci = jax.lax.broadcasted_iota(jnp.int32, (_K, _K), 1)
    beats = jnp.where((vr > v) | ((vr == v) & (ci < ri)), 1.0, 0.0)
    rank = jnp.sum(beats, axis=1, keepdims=True)
    perm = jnp.where(jnp.reshape(rank, (1, _K)) == ri.astype(jnp.float32),
                     1.0, 0.0)
    small = jnp.concatenate([acc_x, acc_y, acc_s, v], axis=1)
    dsrt = jnp.dot(perm, acc, preferred_element_type=jnp.float32)
    msrt = jnp.dot(perm, small, preferred_element_type=jnp.float32)
    scores = 1.0 / (1.0 + jnp.exp(-msrt[:, 3:4]))

    projrow = proj_ref[...]
    dists = []
    for sd in range(4):
        d = dsrt[:, sd * _REG:(sd + 1) * _REG]
        d = d - jnp.max(d, axis=1, keepdims=True)
        e = jnp.exp(d)
        dists.append(jnp.sum(e * projrow, axis=1, keepdims=True)
                     / jnp.sum(e, axis=1, keepdims=True))

    ax = msrt[:, 0:1]
    ay = msrt[:, 1:2]
    ssrt = msrt[:, 2:3]
    x1 = (ax - dists[0]) * ssrt
    y1 = (ay - dists[1]) * ssrt
    x2 = (ax + dists[2]) * ssrt
    y2 = (ay + dists[3]) * ssrt

    x1r = jnp.reshape(x1, (1, _K))
    y1r = jnp.reshape(y1, (1, _K))
    x2r = jnp.reshape(x2, (1, _K))
    y2r = jnp.reshape(y2, (1, _K))
    w = jnp.clip(jnp.minimum(x2, x2r) - jnp.maximum(x1, x1r), 0.0, None)
    h = jnp.clip(jnp.minimum(y2, y2r) - jnp.maximum(y1, y1r), 0.0, None)
    inter = w * h
    area = (x2 - x1) * (y2 - y1)
    union = area + jnp.reshape(area, (1, _K)) - inter + 1e-10
    iou = inter / union
    iou = jnp.where(ri < ci, iou, 0.0)
    keep = (jnp.max(iou, axis=0, keepdims=True) <= _IOU_THR)
    final = jnp.reshape(scores, (1, _K)) * keep.astype(jnp.float32)
    out_ref[0] = jnp.concatenate(
        [x1, y1, x2, y2, jnp.reshape(final, (_K, 1))], axis=1)


@jax.jit
def kernel(pred_dist, cls_logits, anchor_points, stride_tensor, proj):
    b, l, _ = cls_logits.shape
    ch = 400
    nch = l // ch
    m3 = pl.pallas_call(
        _reduce_kernel,
        grid=(nch,),
        in_specs=[pl.BlockSpec((b, ch, _C), lambda i: (0, i, 0))],
        out_specs=pl.BlockSpec((1, b, ch), lambda i: (i, 0, 0)),
        out_shape=jax.ShapeDtypeStruct((nch, b, ch), jnp.float32),
    )(cls_logits)
    m = m3.transpose(1, 0, 2).reshape(b, l)

    sel, slot = pl.pallas_call(
        _select_kernel,
        out_shape=(jax.ShapeDtypeStruct((b, l), jnp.float32),
                   jax.ShapeDtypeStruct((b, l), jnp.float32)),
    )(m)

    out = pl.pallas_call(
        _decode_nms_kernel,
        grid=(b,),
        in_specs=[
            pl.BlockSpec((1, l, 4 * _REG), lambda i: (i, 0, 0)),
            pl.BlockSpec((1, 1, l), lambda i: (i, 0, 0)),
            pl.BlockSpec((1, 1, l), lambda i: (i, 0, 0)),
            pl.BlockSpec((1, 1, l), lambda i: (i, 0, 0)),
            pl.BlockSpec((2, l), lambda i: (0, 0)),
            pl.BlockSpec((1, l), lambda i: (0, 0)),
            pl.BlockSpec((1, _REG), lambda i: (0, 0)),
        ],
        out_specs=pl.BlockSpec((1, _K, 5), lambda i: (i, 0, 0)),
        out_shape=jax.ShapeDtypeStruct((b, _K, 5), jnp.float32),
    )(pred_dist, m.reshape(b, 1, l), sel.reshape(b, 1, l),
      slot.reshape(b, 1, l), anchor_points.T, stride_tensor.reshape(1, l),
      proj.reshape(1, _REG))
    return out
